# baseline (speedup 1.0000x reference)
import functools

import jax
import jax.numpy as jnp
from jax import lax
from jax.experimental import pallas as pl
from jax.experimental.pallas import tpu as pltpu
from jax.experimental.pallas import tpu_sc as plsc

NC, NS, LANES = 2, 16, 16
NW = NC * NS
CH_EL = 32 * 768


def _sc_add(n_el, p_el):
    per_w = n_el // NW
    wpb = p_el // per_w
    nch = per_w // CH_EL

    mesh = plsc.VectorSubcoreMesh(core_axis_name="c", subcore_axis_name="s")

    @functools.partial(
        pl.kernel,
        out_type=jax.ShapeDtypeStruct((n_el,), jnp.float32),
        mesh=mesh,
        scratch_types=[
            pltpu.VMEM((CH_EL,), jnp.float32),
            pltpu.VMEM((CH_EL,), jnp.float32),
        ],
    )
    def sc_kernel(in_hbm, pos_hbm, out_hbm, ibuf, pbuf):
        wid = lax.axis_index("s") * NC + lax.axis_index("c")
        base = wid * per_w
        pos_base = (wid % wpb) * per_w
        for c in range(nch):
            off = base + c * CH_EL
            poff = pos_base + c * CH_EL
            pltpu.sync_copy(in_hbm.at[pl.ds(off, CH_EL)], ibuf)
            pltpu.sync_copy(pos_hbm.at[pl.ds(poff, CH_EL)], pbuf)

            @plsc.parallel_loop(0, CH_EL, LANES, unroll=8)
            def _(i):
                plsc.addupdate(ibuf.at[pl.ds(i, LANES)], pbuf[pl.ds(i, LANES)])

            pltpu.sync_copy(ibuf, out_hbm.at[pl.ds(off, CH_EL)])

    return sc_kernel


def kernel(inputs, pos_table):
    B, S, D = inputs.shape
    in_flat = inputs.astype(jnp.float32).reshape(B * S * D)
    pos_flat = pos_table.reshape(S * D)
    out = _sc_add(in_flat.shape[0], pos_flat.shape[0])(in_flat, pos_flat)
    return out.reshape(B, S, D)


# speedup vs baseline: 1.1134x; 1.1134x over previous
import functools

import jax
import jax.numpy as jnp
from jax import lax
from jax.experimental import pallas as pl
from jax.experimental.pallas import tpu as pltpu
from jax.experimental.pallas import tpu_sc as plsc

NC, NS, LANES = 2, 16, 16
NW = NC * NS
CH_EL = 32 * 768


def _sc_add(n_el, p_el):
    per_w = n_el // NW
    wpb = p_el // per_w
    nch = per_w // CH_EL

    mesh = plsc.VectorSubcoreMesh(core_axis_name="c", subcore_axis_name="s")

    @functools.partial(
        pl.kernel,
        out_type=jax.ShapeDtypeStruct((n_el,), jnp.float32),
        mesh=mesh,
        scratch_types=[
            pltpu.VMEM((2, CH_EL), jnp.float32),
            pltpu.VMEM((2, CH_EL), jnp.float32),
            pltpu.SemaphoreType.DMA,
            pltpu.SemaphoreType.DMA,
        ],
    )
    def sc_kernel(in_hbm, pos_hbm, out_hbm, ibuf, pbuf, lsem, ssem):
        wid = lax.axis_index("s") * NC + lax.axis_index("c")
        base = wid * per_w
        pos_base = (wid % wpb) * per_w

        def load(c):
            off = base + c * CH_EL
            poff = pos_base + c * CH_EL
            slot = c % 2
            return (
                pltpu.async_copy(in_hbm.at[pl.ds(off, CH_EL)], ibuf.at[slot], lsem),
                pltpu.async_copy(pos_hbm.at[pl.ds(poff, CH_EL)], pbuf.at[slot], lsem),
            )

        loads = load(0)
        stores = [None, None]
        for c in range(nch):
            slot = c % 2
            other = (c + 1) % 2
            if c + 1 < nch and stores[other] is not None:
                stores[other].wait()
                stores[other] = None
            next_loads = load(c + 1) if c + 1 < nch else None
            for d in loads:
                d.wait()

            @plsc.parallel_loop(0, CH_EL, LANES, unroll=8)
            def _(i):
                plsc.addupdate(
                    ibuf.at[slot].at[pl.ds(i, LANES)], pbuf[slot, pl.ds(i, LANES)]
                )

            stores[slot] = pltpu.async_copy(
                ibuf.at[slot], out_hbm.at[pl.ds(base + c * CH_EL, CH_EL)], ssem
            )
            loads = next_loads
        for d in stores:
            if d is not None:
                d.wait()

    return sc_kernel


def kernel(inputs, pos_table):
    B, S, D = inputs.shape
    in_flat = inputs.astype(jnp.float32).reshape(B * S * D)
    pos_flat = pos_table.reshape(S * D)
    out = _sc_add(in_flat.shape[0], pos_flat.shape[0])(in_flat, pos_flat)
    return out.reshape(B, S, D)


# speedup vs baseline: 1.7514x; 1.5731x over previous
import functools

import jax
import jax.numpy as jnp
from jax import lax
from jax.experimental import pallas as pl
from jax.experimental.pallas import tpu as pltpu
from jax.experimental.pallas import tpu_sc as plsc

NC, NS, LANES = 2, 16, 16
NW = NC * NS
CH = 32
SC_ROWS = 1024
TC_BS = 1024


def _tc_body(in_ref, pos_ref, out_ref, *, npos):
    i = pl.program_id(0)
    off = (i % npos) * TC_BS
    out_ref[...] = in_ref[...] + pos_ref[pl.ds(off, TC_BS), :]


def _tc_add(flat, pos_table, tc_rows):
    S, D = pos_table.shape
    n_rows = flat.shape[0]
    return pl.pallas_call(
        functools.partial(_tc_body, npos=S // TC_BS),
        grid=(tc_rows // TC_BS,),
        in_specs=[
            pl.BlockSpec((TC_BS, D), lambda i: (i, 0)),
            pl.BlockSpec((S, D), lambda i: (0, 0)),
        ],
        out_specs=pl.BlockSpec((TC_BS, D), lambda i: (i, 0)),
        out_shape=jax.ShapeDtypeStruct((n_rows, D), jnp.float32),
        compiler_params=pltpu.CompilerParams(
            dimension_semantics=("arbitrary",),
        ),
    )(flat, pos_table)


def _sc_add(n_el, p_el, row0, sc_rows, d):
    ch_el = CH * d
    per_w = sc_rows // NW * d
    nch = per_w // ch_el
    s_rows = p_el // d

    mesh = plsc.VectorSubcoreMesh(core_axis_name="c", subcore_axis_name="s")

    @functools.partial(
        pl.kernel,
        out_type=jax.ShapeDtypeStruct((sc_rows * d,), jnp.float32),
        mesh=mesh,
        scratch_types=[
            pltpu.VMEM((2, ch_el), jnp.float32),
            pltpu.VMEM((2, ch_el), jnp.float32),
            pltpu.SemaphoreType.DMA,
            pltpu.SemaphoreType.DMA,
        ],
    )
    def sc_kernel(in_hbm, pos_hbm, out_hbm, ibuf, pbuf, lsem, ssem):
        wid = lax.axis_index("s") * NC + lax.axis_index("c")
        base_row = row0 + wid * (per_w // d)
        base = base_row * d
        out_base = wid * per_w

        def load(c):
            pos_row = lax.rem(base_row + c * CH, s_rows)
            slot = c % 2
            return (
                pltpu.async_copy(
                    in_hbm.at[pl.ds(base + c * ch_el, ch_el)], ibuf.at[slot], lsem
                ),
                pltpu.async_copy(
                    pos_hbm.at[pl.ds(pos_row * d, ch_el)], pbuf.at[slot], lsem
                ),
            )

        loads = load(0)
        stores = [None, None]
        for c in range(nch):
            slot = c % 2
            other = (c + 1) % 2
            if c + 1 < nch and stores[other] is not None:
                stores[other].wait()
                stores[other] = None
            next_loads = load(c + 1) if c + 1 < nch else None
            for dsc in loads:
                dsc.wait()

            @plsc.parallel_loop(0, ch_el, LANES, unroll=8)
            def _(i):
                plsc.addupdate(
                    ibuf.at[slot].at[pl.ds(i, LANES)], pbuf[slot, pl.ds(i, LANES)]
                )

            stores[slot] = pltpu.async_copy(
                ibuf.at[slot], out_hbm.at[pl.ds(out_base + c * ch_el, ch_el)], ssem
            )
            loads = next_loads
        for dsc in stores:
            if dsc is not None:
                dsc.wait()

    return sc_kernel


def kernel(inputs, pos_table):
    B, S, D = inputs.shape
    n_rows = B * S
    flat = inputs.astype(jnp.float32).reshape(n_rows, D)
    tc_rows = n_rows - SC_ROWS

    tc_out = _tc_add(flat, pos_table, tc_rows)
    sc_out = _sc_add(n_rows * D, S * D, tc_rows, SC_ROWS, D)(
        flat.reshape(-1), pos_table.reshape(-1)
    )
    out = lax.dynamic_update_slice(tc_out, sc_out.reshape(SC_ROWS, D), (tc_rows, 0))
    return out.reshape(B, S, D)


# speedup vs baseline: 3.1599x; 1.8043x over previous
import functools

import jax
import jax.numpy as jnp
from jax import lax
from jax.experimental import pallas as pl
from jax.experimental.pallas import tpu as pltpu
from jax.experimental.pallas import tpu_sc as plsc

NC, NS, LANES = 2, 16, 16
NW = NC * NS
CH = 32
SC_ROWS = 1024
TC_BS = 1024


def _tc_body(in_ref, pos_ref, out_ref, *, nsb):
    i = pl.program_id(0)
    off = (i % nsb) * TC_BS
    out_ref[0] = in_ref[0] + pos_ref[pl.ds(off, TC_BS), :]


def _tc_add(inputs, pos_table, n_blocks):
    B, S, D = inputs.shape
    nsb = S // TC_BS
    return pl.pallas_call(
        functools.partial(_tc_body, nsb=nsb),
        grid=(n_blocks,),
        in_specs=[
            pl.BlockSpec((1, TC_BS, D), lambda i, nsb=nsb: (i // nsb, i % nsb, 0)),
            pl.BlockSpec((S, D), lambda i: (0, 0)),
        ],
        out_specs=pl.BlockSpec((1, TC_BS, D), lambda i, nsb=nsb: (i // nsb, i % nsb, 0)),
        out_shape=jax.ShapeDtypeStruct((B, S, D), jnp.float32),
        compiler_params=pltpu.CompilerParams(
            dimension_semantics=("arbitrary",),
        ),
    )(inputs, pos_table)


def _sc_add(B, S, D, batch, seq0, sc_rows):
    rows_per_w = sc_rows // NW
    nch = rows_per_w // CH

    mesh = plsc.VectorSubcoreMesh(core_axis_name="c", subcore_axis_name="s")

    @functools.partial(
        pl.kernel,
        out_type=jax.ShapeDtypeStruct((sc_rows, D), jnp.float32),
        mesh=mesh,
        scratch_types=[
            pltpu.VMEM((2, CH, D), jnp.float32),
            pltpu.VMEM((2, CH, D), jnp.float32),
            pltpu.SemaphoreType.DMA,
            pltpu.SemaphoreType.DMA,
        ],
    )
    def sc_kernel(in_hbm, pos_hbm, out_hbm, ibuf, pbuf, lsem, ssem):
        wid = lax.axis_index("s") * NC + lax.axis_index("c")
        row = seq0 + wid * rows_per_w
        out_row = wid * rows_per_w

        def load(c):
            slot = c % 2
            return (
                pltpu.async_copy(
                    in_hbm.at[batch].at[pl.ds(row + c * CH, CH)], ibuf.at[slot], lsem
                ),
                pltpu.async_copy(
                    pos_hbm.at[pl.ds(row + c * CH, CH)], pbuf.at[slot], lsem
                ),
            )

        loads = load(0)
        stores = [None, None]
        for c in range(nch):
            slot = c % 2
            other = (c + 1) % 2
            if c + 1 < nch and stores[other] is not None:
                stores[other].wait()
                stores[other] = None
            next_loads = load(c + 1) if c + 1 < nch else None
            for dsc in loads:
                dsc.wait()

            for r in range(CH):

                @plsc.parallel_loop(0, D, LANES, unroll=8)
                def _(i):
                    plsc.addupdate(
                        ibuf.at[slot].at[r].at[pl.ds(i, LANES)],
                        pbuf[slot, r, pl.ds(i, LANES)],
                    )

            stores[slot] = pltpu.async_copy(
                ibuf.at[slot], out_hbm.at[pl.ds(out_row + c * CH, CH)], ssem
            )
            loads = next_loads
        for dsc in stores:
            if dsc is not None:
                dsc.wait()

    return sc_kernel


def kernel(inputs, pos_table):
    B, S, D = inputs.shape
    inputs = inputs.astype(jnp.float32)
    n_blocks = (B * S - SC_ROWS) // TC_BS

    tc_out = _tc_add(inputs, pos_table, n_blocks)
    sc_out = _sc_add(B, S, D, B - 1, S - SC_ROWS, SC_ROWS)(inputs, pos_table)
    out = lax.dynamic_update_slice(
        tc_out, sc_out.reshape(1, SC_ROWS, D), (B - 1, S - SC_ROWS, 0)
    )
    return out


# speedup vs baseline: 6.6947x; 2.1186x over previous
import functools

import jax
import jax.numpy as jnp
from jax.experimental import pallas as pl
from jax.experimental.pallas import tpu as pltpu


def _add_body(in_ref, pos_ref, out_ref):
    out_ref[...] = in_ref[...] + pos_ref[...]


def kernel(inputs, pos_table):
    B, S, D = inputs.shape
    flat = inputs.astype(jnp.float32).reshape(B * S, D)
    out = pl.pallas_call(
        _add_body,
        grid=(B,),
        in_specs=[
            pl.BlockSpec((S, D), lambda i: (i, 0)),
            pl.BlockSpec((S, D), lambda i: (0, 0)),
        ],
        out_specs=pl.BlockSpec((S, D), lambda i: (i, 0)),
        out_shape=jax.ShapeDtypeStruct((B * S, D), jnp.float32),
        compiler_params=pltpu.CompilerParams(
            dimension_semantics=("arbitrary",),
        ),
    )(flat, pos_table)
    return out.reshape(B, S, D)
